# baseline (device time: 14835 ns/iter reference)
import jax
import jax.numpy as jnp
from jax import lax
from jax.experimental import pallas as pl
from jax.experimental.pallas import tpu as pltpu

N_DEV = 4


def kernel(A, B):
    m, _ = A.shape
    _, n = B.shape
    chunk = m // N_DEV

    def body(a_ref, b_ref, out_hbm, acc_ref, comm_ref, out_vmem,
             out_sem, send_sems, recv_sems):
        my = lax.axis_index("i")

        barrier_sem = pltpu.get_barrier_semaphore()
        for t_rel in range(1, N_DEV):
            peer = lax.rem(my + t_rel, N_DEV)
            pl.semaphore_signal(
                barrier_sem, inc=1,
                device_id=(peer,), device_id_type=pl.DeviceIdType.MESH,
            )
        pl.semaphore_wait(barrier_sem, N_DEV - 1)

        b = b_ref[...]

        rdmas = []
        for t_rel in (2, 1, 3):
            target = lax.rem(my + t_rel, N_DEV)
            rows = pl.ds(target * chunk, chunk)
            partial = jnp.dot(
                a_ref[rows, :], b, preferred_element_type=jnp.float32
            )
            acc_ref[rows, :] = partial.astype(jnp.bfloat16)
            rdma = pltpu.make_async_remote_copy(
                src_ref=acc_ref.at[rows, :],
                dst_ref=comm_ref.at[N_DEV - 1 - t_rel],
                send_sem=send_sems.at[t_rel - 1],
                recv_sem=recv_sems.at[N_DEV - 1 - t_rel],
                device_id=(target,),
                device_id_type=pl.DeviceIdType.MESH,
            )
            rdma.start()
            rdmas.append(rdma)

        acc = jnp.dot(
            a_ref[pl.ds(my * chunk, chunk), :], b,
            preferred_element_type=jnp.float32,
        )

        for s in (0, 2, 1):
            recv = pltpu.make_async_remote_copy(
                src_ref=acc_ref.at[pl.ds(0, chunk), :],
                dst_ref=comm_ref.at[s],
                send_sem=send_sems.at[0],
                recv_sem=recv_sems.at[s],
                device_id=(my,),
                device_id_type=pl.DeviceIdType.MESH,
            )
            recv.wait_recv()
            acc = acc + comm_ref[s].astype(jnp.float32)

        out_vmem[...] = acc.astype(jnp.bfloat16)
        out_copy = pltpu.make_async_copy(out_vmem, out_hbm, out_sem)
        out_copy.start()

        for rdma in rdmas:
            rdma.wait_send()
        out_copy.wait()

    call = pl.pallas_call(
        body,
        out_shape=jax.ShapeDtypeStruct((chunk, n), jnp.bfloat16),
        in_specs=[
            pl.BlockSpec(memory_space=pltpu.VMEM),
            pl.BlockSpec(memory_space=pltpu.VMEM),
        ],
        out_specs=pl.BlockSpec(memory_space=pl.ANY),
        scratch_shapes=[
            pltpu.VMEM((m, n), jnp.bfloat16),
            pltpu.VMEM((N_DEV - 1, chunk, n), jnp.bfloat16),
            pltpu.VMEM((chunk, n), jnp.bfloat16),
            pltpu.SemaphoreType.DMA(()),
            pltpu.SemaphoreType.DMA((N_DEV - 1,)),
            pltpu.SemaphoreType.DMA((N_DEV - 1,)),
        ],
        compiler_params=pltpu.CompilerParams(collective_id=0),
    )
    return call(A.astype(jnp.bfloat16), B.astype(jnp.bfloat16))


# device time: 14718 ns/iter; 1.0079x vs baseline; 1.0079x over previous
import jax
import jax.numpy as jnp
from jax import lax
from jax.experimental import pallas as pl
from jax.experimental.pallas import tpu as pltpu

N_DEV = 4


def kernel(A, B):
    m, _ = A.shape
    _, n = B.shape
    chunk = m // N_DEV

    def body(a_ref, b_ref, out_hbm, acc_ref, comm_ref, out_vmem,
             out_sem, send_sems, recv_sems):
        my = lax.axis_index("i")

        barrier_sem = pltpu.get_barrier_semaphore()
        for t_rel in range(1, N_DEV):
            peer = lax.rem(my + t_rel, N_DEV)
            pl.semaphore_signal(
                barrier_sem, inc=1,
                device_id=(peer,), device_id_type=pl.DeviceIdType.MESH,
            )
        pl.semaphore_wait(barrier_sem, N_DEV - 1)

        b = b_ref[...]

        rdmas = []
        for t_rel in (2, 1, 3):
            target = lax.rem(my + t_rel, N_DEV)
            rows = pl.ds(target * chunk, chunk)
            partial = jnp.dot(
                a_ref[rows, :].astype(jnp.bfloat16), b,
                preferred_element_type=jnp.float32,
            )
            acc_ref[rows, :] = partial.astype(jnp.bfloat16)
            rdma = pltpu.make_async_remote_copy(
                src_ref=acc_ref.at[rows, :],
                dst_ref=comm_ref.at[N_DEV - 1 - t_rel],
                send_sem=send_sems.at[t_rel - 1],
                recv_sem=recv_sems.at[N_DEV - 1 - t_rel],
                device_id=(target,),
                device_id_type=pl.DeviceIdType.MESH,
            )
            rdma.start()
            rdmas.append(rdma)

        acc = jnp.dot(
            a_ref[pl.ds(my * chunk, chunk), :].astype(jnp.bfloat16), b,
            preferred_element_type=jnp.float32,
        )

        for s in (0, 2, 1):
            recv = pltpu.make_async_remote_copy(
                src_ref=acc_ref.at[pl.ds(0, chunk), :],
                dst_ref=comm_ref.at[s],
                send_sem=send_sems.at[0],
                recv_sem=recv_sems.at[s],
                device_id=(my,),
                device_id_type=pl.DeviceIdType.MESH,
            )
            recv.wait_recv()
            acc = acc + comm_ref[s].astype(jnp.float32)

        out_vmem[...] = acc.astype(jnp.bfloat16)
        out_copy = pltpu.make_async_copy(out_vmem, out_hbm, out_sem)
        out_copy.start()

        for rdma in rdmas:
            rdma.wait_send()
        out_copy.wait()

    call = pl.pallas_call(
        body,
        out_shape=jax.ShapeDtypeStruct((chunk, n), jnp.bfloat16),
        in_specs=[
            pl.BlockSpec(memory_space=pltpu.VMEM),
            pl.BlockSpec(memory_space=pltpu.VMEM),
        ],
        out_specs=pl.BlockSpec(memory_space=pl.ANY),
        scratch_shapes=[
            pltpu.VMEM((m, n), jnp.bfloat16),
            pltpu.VMEM((N_DEV - 1, chunk, n), jnp.bfloat16),
            pltpu.VMEM((chunk, n), jnp.bfloat16),
            pltpu.SemaphoreType.DMA(()),
            pltpu.SemaphoreType.DMA((N_DEV - 1,)),
            pltpu.SemaphoreType.DMA((N_DEV - 1,)),
        ],
        compiler_params=pltpu.CompilerParams(collective_id=0),
    )
    return call(A, B.astype(jnp.bfloat16))
